# baseline (speedup 1.0000x reference)
import jax
import jax.numpy as jnp
from jax import lax
from jax.experimental import pallas as pl
from jax.experimental.pallas import tpu as pltpu
from jax.experimental.pallas import tpu_sc as plsc

_NC, _NS = 2, 16
_NW = _NC * _NS
_C = 128
_BLK = 8000


def _linearize(W):
    V, D = W.shape
    return lax.optimization_barrier(W.reshape(V * D)).reshape(V, D)


def kernel(token_ids, W_embed):
    B, H = token_ids.shape
    V, D = W_embed.shape
    n = B * H
    b_per_w = n // _NW
    n_chunks = b_per_w // _C
    flat_ids = token_ids.reshape(n)
    W_lin = _linearize(W_embed).reshape(V, D)

    mesh = plsc.VectorSubcoreMesh(core_axis_name="c", subcore_axis_name="s")

    @pl.kernel(
        mesh=mesh,
        out_type=jax.ShapeDtypeStruct((n, D), jnp.float32),
        compiler_params=pltpu.CompilerParams(use_tc_tiling_on_sc=False),
        scratch_types=[
            pltpu.VMEM((_C,), jnp.int32),
            pltpu.VMEM((_C, D), jnp.float32),
            pltpu.SemaphoreType.DMA,
        ],
    )
    def gather_kernel(table_hbm, idx_hbm, out_hbm, idx_v, rows_v, sem):
        wid = lax.axis_index("s") * _NC + lax.axis_index("c")
        base = wid * b_per_w

        @pl.loop(0, n_chunks)
        def _(ci):
            off = base + ci * _C
            pltpu.sync_copy(idx_hbm.at[pl.ds(off, _C)], idx_v)
            pltpu.async_copy(table_hbm.at[idx_v], rows_v, sem).wait()
            pltpu.sync_copy(rows_v, out_hbm.at[pl.ds(off, _C)])

    return gather_kernel(W_lin, flat_ids).reshape(B, H, D)


# speedup vs baseline: 1.2407x; 1.2407x over previous
import jax
import jax.numpy as jnp
from jax import lax
from jax.experimental import pallas as pl
from jax.experimental.pallas import tpu as pltpu
from jax.experimental.pallas import tpu_sc as plsc

_NC, _NS = 2, 16
_NW = _NC * _NS
_C = 128


def _sc_gather(table, idx):
    n = idx.shape[0]
    V, D = table.shape
    b_per_w = n // _NW
    n_chunks = b_per_w // _C

    mesh = plsc.VectorSubcoreMesh(core_axis_name="c", subcore_axis_name="s")

    @pl.kernel(
        mesh=mesh,
        out_type=jax.ShapeDtypeStruct((n, D), jnp.float32),
        compiler_params=pltpu.CompilerParams(use_tc_tiling_on_sc=False),
        scratch_types=[
            pltpu.VMEM((_C,), jnp.int32),
            pltpu.VMEM((_C, D), jnp.float32),
            pltpu.SemaphoreType.DMA,
        ],
    )
    def gather_kernel(table_hbm, idx_hbm, out_hbm, idx_v, rows_v, sem):
        wid = lax.axis_index("s") * _NC + lax.axis_index("c")
        base = wid * b_per_w

        @pl.loop(0, n_chunks)
        def _(ci):
            off = base + ci * _C
            pltpu.sync_copy(idx_hbm.at[pl.ds(off, _C)], idx_v)
            pltpu.async_copy(table_hbm.at[idx_v], rows_v, sem).wait()
            pltpu.sync_copy(rows_v, out_hbm.at[pl.ds(off, _C)])

    return gather_kernel(table, idx)


def kernel(token_ids, W_embed):
    B, H = token_ids.shape
    V, D = W_embed.shape
    n = B * H
    Q = H // 2

    W_lin = lax.optimization_barrier(
        W_embed.reshape(V // 2, 2 * D)).reshape(V, D)

    ids_hmajor = token_ids.T.reshape(n)
    u = jnp.arange(n, dtype=jnp.int32)
    q, r = u // (2 * B), u % (2 * B)
    b, j = r // 2, r % 2
    idx = ids_hmajor[(2 * q + j) * B + b]

    G = _sc_gather(W_lin, idx)
    G3 = lax.optimization_barrier(G.reshape(Q, B, 2 * D))
    O = jnp.transpose(G3, (0, 2, 1))
    out = lax.optimization_barrier(O).reshape(H, D, B).transpose(2, 0, 1)
    return out


# speedup vs baseline: 1.5904x; 1.2819x over previous
import jax
import jax.numpy as jnp
from jax import lax
from jax.experimental import pallas as pl
from jax.experimental.pallas import tpu as pltpu
from jax.experimental.pallas import tpu_sc as plsc

_NC, _NS = 2, 16
_NW = _NC * _NS
_C = 128


def _sc_gather(table, idx):
    n = idx.shape[0]
    V, D = table.shape
    b_per_w = n // _NW
    n_chunks = b_per_w // _C

    mesh = plsc.VectorSubcoreMesh(core_axis_name="c", subcore_axis_name="s")

    @pl.kernel(
        mesh=mesh,
        out_type=jax.ShapeDtypeStruct((n, D), jnp.float32),
        compiler_params=pltpu.CompilerParams(use_tc_tiling_on_sc=False),
        scratch_types=[
            pltpu.VMEM((_C,), jnp.int32),
            pltpu.VMEM((_C, D), jnp.float32),
            pltpu.SemaphoreType.DMA,
        ],
    )
    def gather_kernel(table_hbm, idx_hbm, out_hbm, idx_v, rows_v, sem):
        wid = lax.axis_index("s") * _NC + lax.axis_index("c")
        base = wid * b_per_w

        @pl.loop(0, n_chunks)
        def _(ci):
            off = base + ci * _C
            pltpu.sync_copy(idx_hbm.at[pl.ds(off, _C)], idx_v)
            pltpu.async_copy(table_hbm.at[idx_v], rows_v, sem).wait()
            pltpu.sync_copy(rows_v, out_hbm.at[pl.ds(off, _C)])

    return gather_kernel(table, idx)


_RB = 2048


def _pack_block(x_ref, o_ref):
    o_ref[:, 0:64] = x_ref[:, 0:_RB].T
    o_ref[:, 64:128] = x_ref[:, _RB:2 * _RB].T


def _pack_table(Wt):
    D, V = Wt.shape
    grid = (V + 2 * _RB - 1) // (2 * _RB)
    return pl.pallas_call(
        _pack_block,
        grid=(grid,),
        in_specs=[pl.BlockSpec((D, 2 * _RB), lambda i: (0, i))],
        out_specs=pl.BlockSpec((_RB, 2 * D), lambda i: (i, 0)),
        out_shape=jax.ShapeDtypeStruct((grid * _RB, 2 * D), jnp.float32),
    )(Wt)


def kernel(token_ids, W_embed):
    B, H = token_ids.shape
    V, D = W_embed.shape
    n = B * H
    Q = H // 2

    P = _pack_table(W_embed.T)
    W_lin = P.reshape(2 * P.shape[0], D)

    ids_hmajor = token_ids.T.reshape(n)
    u = jnp.arange(n, dtype=jnp.int32)
    q, r = u // (2 * B), u % (2 * B)
    b, j = r // 2, r % 2
    t = ids_hmajor[(2 * q + j) * B + b]
    g, r = t // (2 * _RB), t % (2 * _RB)
    idx = g * (2 * _RB) + 2 * (r % _RB) + r // _RB

    G = _sc_gather(W_lin, idx)
    G3 = lax.optimization_barrier(G.reshape(Q, B, 2 * D))
    O = jnp.transpose(G3, (0, 2, 1))
    out = lax.optimization_barrier(O).reshape(H, D, B).transpose(2, 0, 1)
    return out


# speedup vs baseline: 2.1354x; 1.3427x over previous
import jax
import jax.numpy as jnp
from jax import lax
from jax.experimental import pallas as pl
from jax.experimental.pallas import tpu as pltpu
from jax.experimental.pallas import tpu_sc as plsc

_NC, _NS = 2, 16
_NW = _NC * _NS
_C = 128


_K = 4
_G = _K * _C


def _sc_gather(table, idx):
    n = idx.shape[0]
    V, D = table.shape
    b_per_w = n // _NW
    n_groups = b_per_w // _G
    assert n_groups % 2 == 0

    mesh = plsc.VectorSubcoreMesh(core_axis_name="c", subcore_axis_name="s")

    @pl.kernel(
        mesh=mesh,
        out_type=jax.ShapeDtypeStruct((n, D), jnp.float32),
        compiler_params=pltpu.CompilerParams(use_tc_tiling_on_sc=False),
        scratch_types=[
            pltpu.VMEM((b_per_w,), jnp.int32),
            pltpu.VMEM((2, _G, D), jnp.float32),
            pltpu.SemaphoreType.DMA,
            pltpu.SemaphoreType.DMA,
            pltpu.SemaphoreType.DMA,
            pltpu.SemaphoreType.DMA,
        ],
    )
    def gather_kernel(table_hbm, idx_hbm, out_hbm, idx_v, rows_v,
                      gsem0, gsem1, wsem0, wsem1):
        wid = lax.axis_index("s") * _NC + lax.axis_index("c")
        base = wid * b_per_w
        gsem = (gsem0, gsem1)
        wsem = (wsem0, wsem1)

        def fire_gather(g, b):
            for k in range(_K):
                pltpu.async_copy(
                    table_hbm.at[idx_v.at[pl.ds(g * _G + k * _C, _C)]],
                    rows_v.at[b].at[pl.ds(k * _C, _C)],
                    gsem[b],
                )

        def drain_gather(b):
            pltpu.make_async_copy(
                table_hbm.at[pl.ds(0, _G)], rows_v.at[b], gsem[b]
            ).wait()

        def fire_wb(g, b):
            pltpu.async_copy(
                rows_v.at[b], out_hbm.at[pl.ds(base + g * _G, _G)], wsem[b])

        def drain_wb(b):
            pltpu.make_async_copy(
                table_hbm.at[pl.ds(0, _G)], rows_v.at[b], wsem[b]
            ).wait()

        pltpu.sync_copy(idx_hbm.at[pl.ds(base, b_per_w)], idx_v)

        fire_gather(0, 0)
        fire_gather(1, 1)

        @pl.loop(0, n_groups - 2, step=2)
        def _(g0):
            drain_gather(0)
            fire_wb(g0, 0)
            drain_wb(0)
            fire_gather(g0 + 2, 0)
            drain_gather(1)
            fire_wb(g0 + 1, 1)
            drain_wb(1)
            fire_gather(g0 + 3, 1)

        drain_gather(0)
        fire_wb(n_groups - 2, 0)
        drain_gather(1)
        fire_wb(n_groups - 1, 1)
        drain_wb(0)
        drain_wb(1)

    return gather_kernel(table, idx)


_RB = 2048


def _pack_block(x_ref, o_ref):
    o_ref[:, 0:64] = x_ref[:, 0:_RB].T
    o_ref[:, 64:128] = x_ref[:, _RB:2 * _RB].T


def _pack_table(Wt):
    D, V = Wt.shape
    grid = (V + 2 * _RB - 1) // (2 * _RB)
    return pl.pallas_call(
        _pack_block,
        grid=(grid,),
        in_specs=[pl.BlockSpec((D, 2 * _RB), lambda i: (0, i))],
        out_specs=pl.BlockSpec((_RB, 2 * D), lambda i: (i, 0)),
        out_shape=jax.ShapeDtypeStruct((grid * _RB, 2 * D), jnp.float32),
    )(Wt)


def kernel(token_ids, W_embed):
    B, H = token_ids.shape
    V, D = W_embed.shape
    n = B * H
    Q = H // 2

    P = _pack_table(W_embed.T)
    W_lin = P.reshape(2 * P.shape[0], D)

    ids_hmajor = token_ids.T.reshape(n)
    u = jnp.arange(n, dtype=jnp.int32)
    q, r = u // (2 * B), u % (2 * B)
    b, j = r // 2, r % 2
    t = ids_hmajor[(2 * q + j) * B + b]
    g, r = t // (2 * _RB), t % (2 * _RB)
    idx = g * (2 * _RB) + 2 * (r % _RB) + r // _RB

    G = _sc_gather(W_lin, idx)
    G3 = lax.optimization_barrier(G.reshape(Q, B, 2 * D))
    O = jnp.transpose(G3, (0, 2, 1))
    out = lax.optimization_barrier(O).reshape(H, D, B).transpose(2, 0, 1)
    return out


# speedup vs baseline: 2.3262x; 1.0894x over previous
import jax
import jax.numpy as jnp
from jax import lax
from jax.experimental import pallas as pl
from jax.experimental.pallas import tpu as pltpu
from jax.experimental.pallas import tpu_sc as plsc

_NC, _NS = 2, 16
_NW = _NC * _NS
_C = 128


_K = 4
_G = _K * _C


def _sc_gather(table, idx):
    n = idx.shape[0]
    V, D = table.shape
    b_per_w = n // _NW
    n_groups = b_per_w // _G
    assert n_groups % 2 == 0

    mesh = plsc.VectorSubcoreMesh(core_axis_name="c", subcore_axis_name="s")

    @pl.kernel(
        mesh=mesh,
        out_type=jax.ShapeDtypeStruct((n, D), jnp.float32),
        compiler_params=pltpu.CompilerParams(use_tc_tiling_on_sc=False),
        scratch_types=[
            pltpu.VMEM((b_per_w,), jnp.int32),
            pltpu.VMEM((2, _G, D), jnp.float32),
            pltpu.SemaphoreType.DMA,
            pltpu.SemaphoreType.DMA,
            pltpu.SemaphoreType.DMA,
            pltpu.SemaphoreType.DMA,
        ],
    )
    def gather_kernel(table_hbm, idx_hbm, out_hbm, idx_v, rows_v,
                      gsem0, gsem1, wsem0, wsem1):
        wid = lax.axis_index("s") * _NC + lax.axis_index("c")
        base = wid * b_per_w
        gsem = (gsem0, gsem1)
        wsem = (wsem0, wsem1)

        def fire_gather(g, b):
            for k in range(_K):
                pltpu.async_copy(
                    table_hbm.at[idx_v.at[pl.ds(g * _G + k * _C, _C)]],
                    rows_v.at[b].at[pl.ds(k * _C, _C)],
                    gsem[b],
                )

        def drain_gather(b):
            pltpu.make_async_copy(
                table_hbm.at[pl.ds(0, _G)], rows_v.at[b], gsem[b]
            ).wait()

        def fire_wb(g, b):
            pltpu.async_copy(
                rows_v.at[b], out_hbm.at[pl.ds(base + g * _G, _G)], wsem[b])

        def drain_wb(b):
            pltpu.make_async_copy(
                table_hbm.at[pl.ds(0, _G)], rows_v.at[b], wsem[b]
            ).wait()

        pltpu.sync_copy(idx_hbm.at[pl.ds(base, b_per_w)], idx_v)

        fire_gather(0, 0)
        fire_gather(1, 1)

        @pl.loop(0, n_groups - 2, step=2)
        def _(g0):
            drain_gather(0)
            fire_wb(g0, 0)
            drain_wb(0)
            fire_gather(g0 + 2, 0)
            drain_gather(1)
            fire_wb(g0 + 1, 1)
            drain_wb(1)
            fire_gather(g0 + 3, 1)

        drain_gather(0)
        fire_wb(n_groups - 2, 0)
        drain_gather(1)
        fire_wb(n_groups - 1, 1)
        drain_wb(0)
        drain_wb(1)

    return gather_kernel(table, idx)


_RB = 4096


def _pack_block(x_ref, o_ref):
    o_ref[...] = jnp.concatenate(
        [x_ref[:, 0:_RB].T, x_ref[:, _RB:2 * _RB].T], axis=1)


def _pack_table(Wt):
    D, V = Wt.shape
    grid = (V + 2 * _RB - 1) // (2 * _RB)
    return pl.pallas_call(
        _pack_block,
        grid=(grid,),
        in_specs=[pl.BlockSpec((D, 2 * _RB), lambda i: (0, i))],
        out_specs=pl.BlockSpec((_RB, 2 * D), lambda i: (i, 0)),
        out_shape=jax.ShapeDtypeStruct((grid * _RB, 2 * D), jnp.float32),
    )(Wt)


def kernel(token_ids, W_embed):
    B, H = token_ids.shape
    V, D = W_embed.shape
    n = B * H
    Q = H // 2

    P = _pack_table(W_embed.T)
    W_lin = P.reshape(2 * P.shape[0], D)

    ids_hmajor = token_ids.T.reshape(n)
    u = jnp.arange(n, dtype=jnp.int32)
    q, r = u // (2 * B), u % (2 * B)
    b, j = r // 2, r % 2
    t = ids_hmajor[(2 * q + j) * B + b]
    g, r = t // (2 * _RB), t % (2 * _RB)
    idx = g * (2 * _RB) + 2 * (r % _RB) + r // _RB

    G = _sc_gather(W_lin, idx)
    G3 = lax.optimization_barrier(G.reshape(Q, B, 2 * D))
    O = jnp.transpose(G3, (0, 2, 1))
    out = lax.optimization_barrier(O).reshape(H, D, B).transpose(2, 0, 1)
    return out
